# Initial kernel scaffold; baseline (speedup 1.0000x reference)
import jax
import jax.numpy as jnp
from jax.experimental import pallas as pl


def kernel(token_ids, weight):
    raise NotImplementedError("write your pallas kernel here")


# speedup vs baseline: 1.8445x; 1.8445x over previous
import functools

import jax
import jax.numpy as jnp
from jax import lax
from jax.experimental import pallas as pl
from jax.experimental.pallas import tpu as pltpu
from jax.experimental.pallas import tpu_sc as plsc

_NUM_CORES = 2
_NUM_SUBCORES = 16
_CHUNK = 1024


@functools.cache
def _build_gather(B, D):
    nw = _NUM_CORES * _NUM_SUBCORES
    assert B % (nw * _CHUNK) == 0
    b_per_w = B // nw
    n_chunks = b_per_w // _CHUNK
    mesh = plsc.VectorSubcoreMesh(core_axis_name="c", subcore_axis_name="s")

    def body(table_hbm, idx_hbm, out_hbm, idx_v, rows_v, sem):
        wid = lax.axis_index("s") * _NUM_CORES + lax.axis_index("c")
        base = wid * b_per_w

        @pl.loop(0, n_chunks)
        def _chunk(g):
            off = base + g * _CHUNK
            pltpu.sync_copy(idx_hbm.at[pl.ds(off, _CHUNK)], idx_v)
            pltpu.async_copy(table_hbm.at[idx_v], rows_v, sem).wait()
            pltpu.sync_copy(rows_v, out_hbm.at[pl.ds(off, _CHUNK)])

    return pl.kernel(
        body,
        out_type=jax.ShapeDtypeStruct((B, D), jnp.float32),
        mesh=mesh,
        compiler_params=pltpu.CompilerParams(use_tc_tiling_on_sc=False),
        scratch_types=[
            pltpu.VMEM((_CHUNK,), jnp.int32),
            pltpu.VMEM((_CHUNK, D), jnp.float32),
            pltpu.SemaphoreType.DMA,
        ],
    )


def kernel(token_ids, weight):
    lead = token_ids.shape
    d = weight.shape[1]
    b = 1
    for s in lead:
        b *= s
    idx = token_ids.reshape((b,)).astype(jnp.int32)
    out = _build_gather(b, d)(weight, idx)
    return out.reshape(lead + (d,))


# speedup vs baseline: 1.8613x; 1.0091x over previous
import functools

import jax
import jax.numpy as jnp
from jax import lax
from jax.experimental import pallas as pl
from jax.experimental.pallas import tpu as pltpu
from jax.experimental.pallas import tpu_sc as plsc

_NUM_CORES = 2
_NUM_SUBCORES = 16
_CHUNK = 800
_NBUF = 2


@functools.cache
def _build_gather(B, D):
    nw = _NUM_CORES * _NUM_SUBCORES
    assert B % (nw * _CHUNK * _NBUF) == 0
    b_per_w = B // nw
    n_chunks = b_per_w // _CHUNK
    mesh = plsc.VectorSubcoreMesh(core_axis_name="c", subcore_axis_name="s")

    def body(table_hbm, idx_hbm, out_hbm, idx_v, rows_v, *sems):
        wid = lax.axis_index("s") * _NUM_CORES + lax.axis_index("c")
        base = wid * b_per_w

        def issue(g, b):
            off = base + g * _CHUNK
            pltpu.sync_copy(idx_hbm.at[pl.ds(off, _CHUNK)], idx_v.at[b])
            pltpu.async_copy(table_hbm.at[idx_v.at[b]], rows_v.at[b], sems[b])

        def drain_store(g, b):
            pltpu.make_async_copy(
                table_hbm.at[idx_v.at[b]], rows_v.at[b], sems[b]).wait()
            off = base + g * _CHUNK
            pltpu.sync_copy(rows_v.at[b], out_hbm.at[pl.ds(off, _CHUNK)])

        for b in range(_NBUF):
            issue(b, b)

        @pl.loop(0, n_chunks - _NBUF, step=_NBUF)
        def _main(go):
            for b in range(_NBUF):
                drain_store(go + b, b)
                issue(go + b + _NBUF, b)

        for b in range(_NBUF):
            drain_store(n_chunks - _NBUF + b, b)

    return pl.kernel(
        body,
        out_type=jax.ShapeDtypeStruct((B, D), jnp.float32),
        mesh=mesh,
        compiler_params=pltpu.CompilerParams(use_tc_tiling_on_sc=False),
        scratch_types=[
            pltpu.VMEM((_NBUF, _CHUNK), jnp.int32),
            pltpu.VMEM((_NBUF, _CHUNK, D), jnp.float32),
        ] + [pltpu.SemaphoreType.DMA] * _NBUF,
    )


def kernel(token_ids, weight):
    lead = token_ids.shape
    d = weight.shape[1]
    b = 1
    for s in lead:
        b *= s
    idx = token_ids.reshape((b,)).astype(jnp.int32)
    out = _build_gather(b, d)(weight, idx)
    return out.reshape(lead + (d,))
